# baseline (speedup 1.0000x reference)
import functools

import jax
import jax.numpy as jnp
from jax import lax
from jax.experimental import pallas as pl
from jax.experimental.pallas import tpu as pltpu
from jax.experimental.pallas import tpu_sc as plsc

_N = 50000
_NP = 50176
_E = 3200000
_CH = 128
_NCH = _E // _CH
_K = 8
_G = 128
_NC, _NS = 2, 16
_NW = _NC * _NS
_GRP = 98
_WCH = _GRP * _K
_MAING = (_NCH - 31 * _WCH) // _K
_PADCH = _NW * _WCH - _NCH
_RPT = _NP // _NS
_NB = 49
_BLK = 1024
_NR = _NP // 128

_F32 = jnp.float32


def _mesh():
    return plsc.VectorSubcoreMesh(
        core_axis_name="c", subcore_axis_name="s",
        num_cores=_NC, num_subcores=_NS)


def _sc_params(layout_passes=True):
    return pltpu.CompilerParams(use_tc_tiling_on_sc=False,
                                needs_layout_passes=layout_passes)


def _zero_rows(ref, nrows):
    def body(i, _):
        ref[i] = jnp.zeros((16,), _F32)
        return 0
    lax.fori_loop(0, nrows, body, 0)


def _zero_flat(ref, n):
    def body(i, _):
        ref[pl.ds(i * 16, 16)] = jnp.zeros((16,), _F32)
        return 0
    lax.fori_loop(0, n // 16, body, 0)


def _edge_loader(er_hbm, pad_hbm, row, base, is31, si_v, di_v, sem_i):
    def fire_idx(g, b):
        pad = jnp.logical_and(is31, g >= _MAING)

        @pl.when(jnp.logical_not(pad))
        def _():
            cb = pl.ds(base + g * _K, _K)
            pltpu.async_copy(er_hbm.at[0, cb, :], si_v.at[b], sem_i[b])
            pltpu.async_copy(er_hbm.at[1, cb, :], di_v.at[b], sem_i[b])

        @pl.when(pad)
        def _():
            pb = pl.ds((g - _MAING) * _K, _K)
            pltpu.async_copy(pad_hbm.at[pb, :], si_v.at[b], sem_i[b])
            pltpu.async_copy(pad_hbm.at[pb, :], di_v.at[b], sem_i[b])

    def wait_idx(b):
        cb = pl.ds(0, _K)
        pltpu.make_async_copy(er_hbm.at[0, cb, :], si_v.at[b],
                              sem_i[b]).wait()
        pltpu.make_async_copy(er_hbm.at[1, cb, :], di_v.at[b],
                              sem_i[b]).wait()
    del row
    return fire_idx, wait_idx


def _sc_hist(er, padch):
    @functools.partial(
        pl.kernel,
        out_type=jax.ShapeDtypeStruct((_NC * _NP,), _F32),
        mesh=_mesh(),
        compiler_params=_sc_params(),
        scratch_types=[
            pltpu.VMEM((2, _K, _CH), jnp.int32),
            pltpu.VMEM((_CH,), _F32),
            pltpu.VMEM((_RPT,), _F32),
            pltpu.VMEM_SHARED((_NP,), _F32),
            pltpu.SemaphoreType.DMA,
            pltpu.SemaphoreType.DMA,
            pltpu.SemaphoreType.DMA,
        ],
    )
    def body(er_hbm, pad_hbm, out_hbm, di_v, ones_v, stage_v, acc_sh,
             s_i0, s_i1, s_s):
        c = lax.axis_index("c")
        s = lax.axis_index("s")
        w = c * _NS + s
        base = w * _WCH
        is31 = w == _NW - 1
        sl = pl.ds(s * _RPT, _RPT)
        _zero_flat(stage_v, _RPT)
        for i in range(_CH // 16):
            ones_v[pl.ds(i * 16, 16)] = jnp.ones((16,), _F32)
        pltpu.sync_copy(stage_v, acc_sh.at[sl])
        plsc.subcore_barrier()

        sem_i = (s_i0, s_i1)

        def fire_idx(g, b):
            pad = jnp.logical_and(is31, g >= _MAING)

            @pl.when(jnp.logical_not(pad))
            def _():
                pltpu.async_copy(er_hbm.at[1, pl.ds(base + g * _K, _K), :],
                                 di_v.at[b], sem_i[b])

            @pl.when(pad)
            def _():
                pltpu.async_copy(pad_hbm.at[pl.ds((g - _MAING) * _K, _K), :],
                                 di_v.at[b], sem_i[b])

        def wait_idx(b):
            pltpu.make_async_copy(er_hbm.at[1, pl.ds(0, _K), :], di_v.at[b],
                                  sem_i[b]).wait()

        def fire_sca(b):
            for j in range(_K):
                pltpu.async_copy(ones_v, acc_sh.at[di_v.at[b, j]], s_s,
                                 add=True)

        def wait_sca(b):
            for j in range(_K):
                pltpu.make_async_copy(
                    ones_v, acc_sh.at[di_v.at[b, j]], s_s).wait()

        fire_idx(0, 0)
        fire_idx(1, 1)
        wait_idx(0)
        fire_sca(0)

        def pair(it, _):
            g = 2 * it + 1
            wait_sca(0)
            fire_idx(g + 1, 0)
            wait_idx(1)
            fire_sca(1)
            wait_sca(1)
            fire_idx(g + 2, 1)
            wait_idx(0)
            fire_sca(0)
            return 0
        lax.fori_loop(0, (_GRP - 2) // 2, pair, 0)

        wait_sca(0)
        wait_idx(1)
        fire_sca(1)
        wait_sca(1)

        plsc.subcore_barrier()
        pltpu.sync_copy(acc_sh.at[sl], stage_v)
        pltpu.sync_copy(stage_v, out_hbm.at[pl.ds(c * _NP + s * _RPT, _RPT)])

    return body(er, padch)


def _rsqrt16(d):
    bits = plsc.bitcast(d, jnp.int32)
    y = plsc.bitcast(
        jnp.full((16,), 0x5F3759DF, jnp.int32)
        - lax.shift_right_logical(bits, jnp.full((16,), 1, jnp.int32)),
        _F32)
    hd = 0.5 * d
    for _ in range(3):
        y = y * (1.5 - hd * y * y)
    return y


def _sc_scalar(er, padch, cnt, xflat):
    @functools.partial(
        pl.kernel,
        out_type=[jax.ShapeDtypeStruct((_NC * _NP,), _F32),
                  jax.ShapeDtypeStruct((_NP,), _F32)],
        mesh=_mesh(),
        compiler_params=_sc_params(layout_passes=False),
        scratch_types=[
            pltpu.VMEM((2, _K, _CH), jnp.int32),
            pltpu.VMEM((2, _K, _CH), jnp.int32),
            pltpu.VMEM((2, _K, _CH), _F32),
            pltpu.VMEM((_RPT,), _F32),
            pltpu.VMEM((_RPT,), _F32),
            pltpu.VMEM((_RPT,), _F32),
            pltpu.VMEM_SHARED((_NP,), _F32),
            pltpu.VMEM_SHARED((_NP,), _F32),
            pltpu.SemaphoreType.DMA,
            pltpu.SemaphoreType.DMA,
            pltpu.SemaphoreType.DMA,
            pltpu.SemaphoreType.DMA,
            pltpu.SemaphoreType.DMA,
        ],
    )
    def body(er_hbm, pad_hbm, cnt_hbm, x_hbm, out_hbm, dis_hbm,
             si_v, di_v, vals_v, stage_v, ca_v, cb_v, utab_sh, acc_sh,
             s_i0, s_i1, s_g0, s_g1, s_s):
        c = lax.axis_index("c")
        s = lax.axis_index("s")
        w = c * _NS + s
        base = w * _WCH
        is31 = w == _NW - 1
        sl = pl.ds(s * _RPT, _RPT)
        pltpu.sync_copy(cnt_hbm.at[sl], ca_v)
        pltpu.sync_copy(cnt_hbm.at[pl.ds(_NP + s * _RPT, _RPT)], cb_v)
        pltpu.sync_copy(x_hbm.at[sl], stage_v)

        def dcomp(i, _):
            ix = pl.ds(i * 16, 16)
            deg = ca_v[ix] + cb_v[ix] + 1.0
            dis = _rsqrt16(deg)
            ca_v[ix] = dis
            cb_v[ix] = dis * stage_v[ix]
            return 0
        lax.fori_loop(0, _RPT // 16, dcomp, 0)

        pltpu.sync_copy(cb_v, utab_sh.at[sl])

        @pl.when(c == 0)
        def _():
            pltpu.sync_copy(ca_v, dis_hbm.at[sl])

        _zero_flat(stage_v, _RPT)
        pltpu.sync_copy(stage_v, acc_sh.at[sl])
        plsc.subcore_barrier()

        sem_i = (s_i0, s_i1)
        sem_g = (s_g0, s_g1)
        fire_idx, wait_idx = _edge_loader(
            er_hbm, pad_hbm, 0, base, is31, si_v, di_v, sem_i)

        def fire_gat(b):
            for j in range(_K):
                pltpu.async_copy(
                    utab_sh.at[si_v.at[b, j]], vals_v.at[b, j], sem_g[b])

        def wait_gat(b):
            for j in range(_K):
                pltpu.make_async_copy(
                    utab_sh.at[si_v.at[b, j]], vals_v.at[b, j],
                    sem_g[b]).wait()

        def fire_sca(b):
            for j in range(_K):
                pltpu.async_copy(
                    vals_v.at[b, j], acc_sh.at[di_v.at[b, j]], s_s, add=True)

        def wait_sca(b):
            for j in range(_K):
                pltpu.make_async_copy(
                    vals_v.at[b, j], acc_sh.at[di_v.at[b, j]], s_s).wait()

        fire_idx(0, 0)
        wait_idx(0)
        fire_gat(0)
        fire_idx(1, 1)
        wait_gat(0)
        fire_sca(0)
        wait_idx(1)
        fire_gat(1)

        def steady(g, b):
            nb = 1 - b
            wait_sca(nb)
            fire_idx(g + 1, nb)
            wait_gat(b)
            fire_sca(b)
            wait_idx(nb)
            fire_gat(nb)

        def pair(it, _):
            g = 2 * it + 1
            steady(g, 1)
            steady(g + 1, 0)
            return 0
        lax.fori_loop(0, (_GRP - 2) // 2, pair, 0)

        wait_sca(0)
        wait_gat(1)
        fire_sca(1)
        wait_sca(1)

        plsc.subcore_barrier()
        pltpu.sync_copy(acc_sh.at[sl], stage_v)
        pltpu.sync_copy(stage_v, out_hbm.at[pl.ds(c * _NP + s * _RPT, _RPT)])

    return body(er, padch, cnt, xflat)


def _sc_vec(er, padch, q):
    @functools.partial(
        pl.kernel,
        out_type=jax.ShapeDtypeStruct((_NC * _NP, 16), _F32),
        mesh=_mesh(),
        compiler_params=_sc_params(),
        scratch_types=[
            pltpu.VMEM((2, _K, _CH), jnp.int32),
            pltpu.VMEM((2, _K, _CH), jnp.int32),
            pltpu.VMEM((2, _K * _CH, 16), _F32),
            pltpu.VMEM((_RPT // 4, 16), _F32),
            pltpu.VMEM_SHARED((_NP, 16), _F32),
            pltpu.SemaphoreType.DMA,
            pltpu.SemaphoreType.DMA,
            pltpu.SemaphoreType.DMA,
            pltpu.SemaphoreType.DMA,
            pltpu.SemaphoreType.DMA,
        ],
    )
    def body(er_hbm, pad_hbm, q_hbm, out_hbm,
             si_v, di_v, rows_v, stage_v, acc_sh,
             s_i0, s_i1, s_g0, s_g1, s_s):
        c = lax.axis_index("c")
        s = lax.axis_index("s")
        w = c * _NS + s
        base = w * _WCH
        is31 = w == _NW - 1
        qtr = _RPT // 4
        _zero_rows(stage_v, qtr)
        for r in range(4):
            pltpu.sync_copy(
                stage_v, acc_sh.at[pl.ds(s * _RPT + r * qtr, qtr)])
        plsc.subcore_barrier()

        sem_i = (s_i0, s_i1)
        sem_g = (s_g0, s_g1)
        fire_idx, wait_idx = _edge_loader(
            er_hbm, pad_hbm, 0, base, is31, si_v, di_v, sem_i)

        def fire_gat(b):
            for j in range(_K):
                pltpu.async_copy(
                    q_hbm.at[si_v.at[b, j]],
                    rows_v.at[b, pl.ds(j * _CH, _CH), :], sem_g[b])

        def wait_gat(b):
            for j in range(_K):
                pltpu.make_async_copy(
                    q_hbm.at[si_v.at[b, j]],
                    rows_v.at[b, pl.ds(j * _CH, _CH), :], sem_g[b]).wait()

        def fire_sca(b):
            for j in range(_K):
                pltpu.async_copy(
                    rows_v.at[b, pl.ds(j * _CH, _CH), :],
                    acc_sh.at[di_v.at[b, j]], s_s, add=True)

        def wait_sca(b):
            for j in range(_K):
                pltpu.make_async_copy(
                    rows_v.at[b, pl.ds(j * _CH, _CH), :],
                    acc_sh.at[di_v.at[b, j]], s_s).wait()

        fire_idx(0, 0)
        wait_idx(0)
        fire_gat(0)
        fire_idx(1, 1)
        wait_gat(0)
        fire_sca(0)
        wait_idx(1)
        fire_gat(1)

        def steady(g, b):
            nb = 1 - b
            wait_sca(nb)
            fire_idx(g + 1, nb)
            wait_gat(b)
            fire_sca(b)
            wait_idx(nb)
            fire_gat(nb)

        def pair(it, _):
            g = 2 * it + 1
            steady(g, 1)
            steady(g + 1, 0)
            return 0
        lax.fori_loop(0, (_GRP - 2) // 2, pair, 0)

        wait_sca(0)
        wait_gat(1)
        fire_sca(1)
        wait_sca(1)

        plsc.subcore_barrier()
        for r in range(4):
            pltpu.sync_copy(
                acc_sh.at[pl.ds(s * _RPT + r * qtr, qtr)], stage_v)
            pltpu.sync_copy(
                stage_v,
                out_hbm.at[pl.ds(c * _NP + s * _RPT + r * qtr, qtr), :])

    return body(er, padch, q)


def _tc_prep2(s1p, dis, xp, W1, b1, W2):
    def body(s1a_ref, s1b_ref, dis_ref, x_ref, w1_ref, b1_ref, w2_ref, q_ref):
        dis = dis_ref[...]
        p = dis * (s1a_ref[...] + s1b_ref[...]) + dis * dis * x_ref[...]
        h1 = jnp.maximum(p * w1_ref[...] + b1_ref[...], 0.0)
        t = jnp.dot(h1, w2_ref[...], preferred_element_type=_F32)
        q_ref[...] = dis * t

    grid = (_NB,)
    return pl.pallas_call(
        body,
        grid=grid,
        in_specs=[
            pl.BlockSpec((_BLK, 1), lambda i: (i, 0)),
            pl.BlockSpec((_BLK, 1), lambda i: (_NB + i, 0)),
            pl.BlockSpec((_BLK, 1), lambda i: (i, 0)),
            pl.BlockSpec((_BLK, 1), lambda i: (i, 0)),
            pl.BlockSpec((1, 64), lambda i: (0, 0)),
            pl.BlockSpec((1, 64), lambda i: (0, 0)),
            pl.BlockSpec((64, 16), lambda i: (0, 0)),
        ],
        out_specs=pl.BlockSpec((_BLK, 16), lambda i: (i, 0)),
        out_shape=jax.ShapeDtypeStruct((_NP, 16), _F32),
    )(s1p, s1p, dis, xp, W1, b1, W2)


def _tc_final(s2p, q, dis, batch2d, b2, W3, b3):
    def body(s2a_ref, s2b_ref, q_ref, dis_ref, b_ref, b2_ref, w3_ref, b3_ref,
             out_ref, acc_ref):
        i = pl.program_id(0)

        @pl.when(i == 0)
        def _():
            acc_ref[...] = jnp.zeros((_G, 2), _F32)

        dis = dis_ref[...]
        o2 = dis * (s2a_ref[...] + s2b_ref[...] + q_ref[...]) + b2_ref[...]
        h2 = jnp.maximum(o2, 0.0)
        h3 = jnp.dot(h2, w3_ref[...], preferred_element_type=_F32) \
            + b3_ref[...]
        gids = lax.broadcasted_iota(jnp.int32, (_BLK, _G), 1)
        oh = (b_ref[...] == gids).astype(_F32)
        hcat = jnp.concatenate(
            [h3, jnp.ones((_BLK, 1), _F32)], axis=1)
        acc_ref[...] += lax.dot_general(
            oh, hcat, (((0,), (0,)), ((), ())),
            preferred_element_type=_F32)

        @pl.when(i == _NB - 1)
        def _():
            a = acc_ref[...]
            out_ref[...] = a[:, 0:1] / jnp.maximum(a[:, 1:2], 1.0)

    grid = (_NB,)
    return pl.pallas_call(
        body,
        grid=grid,
        in_specs=[
            pl.BlockSpec((_BLK, 16), lambda i: (i, 0)),
            pl.BlockSpec((_BLK, 16), lambda i: (_NB + i, 0)),
            pl.BlockSpec((_BLK, 16), lambda i: (i, 0)),
            pl.BlockSpec((_BLK, 1), lambda i: (i, 0)),
            pl.BlockSpec((_BLK, 1), lambda i: (i, 0)),
            pl.BlockSpec((1, 16), lambda i: (0, 0)),
            pl.BlockSpec((16, 1), lambda i: (0, 0)),
            pl.BlockSpec((1, 1), lambda i: (0, 0)),
        ],
        out_specs=pl.BlockSpec((_G, 1), lambda i: (0, 0)),
        out_shape=jax.ShapeDtypeStruct((_G, 1), _F32),
        scratch_shapes=[pltpu.VMEM((_G, 2), _F32)],
    )(s2p, s2p, q, dis, batch2d, b2, W3, b3)


def kernel(x, edge_index, batch, W1, b1, W2, b2, W3, b3):
    pad = _NP - _N
    xp = jnp.pad(x[:, 0], (0, pad)).reshape(_NR, 128)
    batch2d = jnp.pad(batch, (0, pad), constant_values=-1)

    er = edge_index.reshape(2, _NCH, _CH)
    padch = (_N + (jnp.arange(_PADCH * _CH, dtype=jnp.int32) % pad)
             ).reshape(_PADCH, _CH)

    cntp = _sc_hist(er, padch)
    x_c = xp.reshape(_NP, 1)

    s1_flat, dis = _sc_scalar(er, padch, cntp, xp.reshape(_NP))
    s1p = s1_flat.reshape(_NC * _NP, 1)
    dis_c = dis.reshape(_NP, 1)
    q = _tc_prep2(s1p, dis_c, x_c, W1, b1.reshape(1, 64), W2)

    s2p = _sc_vec(er, padch, q)
    out = _tc_final(s2p, q, dis_c, batch2d.reshape(_NP, 1),
                    b2.reshape(1, 16), W3, b3.reshape(1, 1))
    return out


# speedup vs baseline: 1.0005x; 1.0005x over previous
import functools

import jax
import jax.numpy as jnp
from jax import lax
from jax.experimental import pallas as pl
from jax.experimental.pallas import tpu as pltpu
from jax.experimental.pallas import tpu_sc as plsc

_N = 50000
_NP = 50176
_E = 3200000
_CH = 128
_NCH = _E // _CH
_K = 8
_G = 128
_NC, _NS = 2, 16
_NW = _NC * _NS
_GRP = 98
_WCH = _GRP * _K
_MAING = (_NCH - 31 * _WCH) // _K
_PADCH = _NW * _WCH - _NCH
_RPT = _NP // _NS
_NB = 49
_BLK = 1024
_NR = _NP // 128

_F32 = jnp.float32


def _mesh():
    return plsc.VectorSubcoreMesh(
        core_axis_name="c", subcore_axis_name="s",
        num_cores=_NC, num_subcores=_NS)


def _sc_params(layout_passes=True):
    return pltpu.CompilerParams(use_tc_tiling_on_sc=False,
                                needs_layout_passes=layout_passes)


def _zero_rows(ref, nrows):
    def body(i, _):
        ref[i] = jnp.zeros((16,), _F32)
        return 0
    lax.fori_loop(0, nrows, body, 0)


def _zero_flat(ref, n):
    def body(i, _):
        ref[pl.ds(i * 16, 16)] = jnp.zeros((16,), _F32)
        return 0
    lax.fori_loop(0, n // 16, body, 0)


def _edge_loader(er_hbm, pad_hbm, row, base, is31, si_v, di_v, sem_i):
    def fire_idx(g, b):
        pad = jnp.logical_and(is31, g >= _MAING)

        @pl.when(jnp.logical_not(pad))
        def _():
            cb = pl.ds(base + g * _K, _K)
            pltpu.async_copy(er_hbm.at[0, cb, :], si_v.at[b], sem_i[b])
            pltpu.async_copy(er_hbm.at[1, cb, :], di_v.at[b], sem_i[b])

        @pl.when(pad)
        def _():
            pb = pl.ds((g - _MAING) * _K, _K)
            pltpu.async_copy(pad_hbm.at[pb, :], si_v.at[b], sem_i[b])
            pltpu.async_copy(pad_hbm.at[pb, :], di_v.at[b], sem_i[b])

    def wait_idx(b):
        cb = pl.ds(0, _K)
        pltpu.make_async_copy(er_hbm.at[0, cb, :], si_v.at[b],
                              sem_i[b]).wait()
        pltpu.make_async_copy(er_hbm.at[1, cb, :], di_v.at[b],
                              sem_i[b]).wait()
    del row
    return fire_idx, wait_idx


def _sc_hist(er, padch):
    @functools.partial(
        pl.kernel,
        out_type=jax.ShapeDtypeStruct((_NC * _NP,), _F32),
        mesh=_mesh(),
        compiler_params=_sc_params(),
        scratch_types=[
            pltpu.VMEM((2, _K, _CH), jnp.int32),
            pltpu.VMEM((_CH,), _F32),
            pltpu.VMEM((_RPT,), _F32),
            pltpu.VMEM_SHARED((_NP,), _F32),
            pltpu.SemaphoreType.DMA,
            pltpu.SemaphoreType.DMA,
            pltpu.SemaphoreType.DMA,
        ],
    )
    def body(er_hbm, pad_hbm, out_hbm, di_v, ones_v, stage_v, acc_sh,
             s_i0, s_i1, s_s):
        c = lax.axis_index("c")
        s = lax.axis_index("s")
        w = c * _NS + s
        base = w * _WCH
        is31 = w == _NW - 1
        sl = pl.ds(s * _RPT, _RPT)
        _zero_flat(stage_v, _RPT)
        for i in range(_CH // 16):
            ones_v[pl.ds(i * 16, 16)] = jnp.ones((16,), _F32)
        pltpu.sync_copy(stage_v, acc_sh.at[sl])
        plsc.subcore_barrier()

        sem_i = (s_i0, s_i1)

        def fire_idx(g, b):
            pad = jnp.logical_and(is31, g >= _MAING)

            @pl.when(jnp.logical_not(pad))
            def _():
                pltpu.async_copy(er_hbm.at[1, pl.ds(base + g * _K, _K), :],
                                 di_v.at[b], sem_i[b])

            @pl.when(pad)
            def _():
                pltpu.async_copy(pad_hbm.at[pl.ds((g - _MAING) * _K, _K), :],
                                 di_v.at[b], sem_i[b])

        def wait_idx(b):
            pltpu.make_async_copy(er_hbm.at[1, pl.ds(0, _K), :], di_v.at[b],
                                  sem_i[b]).wait()

        def fire_sca(b):
            for j in range(_K):
                pltpu.async_copy(ones_v, acc_sh.at[di_v.at[b, j]], s_s,
                                 add=True)

        def wait_sca(b):
            for j in range(_K):
                pltpu.make_async_copy(
                    ones_v, acc_sh.at[di_v.at[b, j]], s_s).wait()

        fire_idx(0, 0)
        fire_idx(1, 1)
        wait_idx(0)
        fire_sca(0)

        def pair(it, _):
            g = 2 * it + 1
            wait_sca(0)
            fire_idx(g + 1, 0)
            wait_idx(1)
            fire_sca(1)
            wait_sca(1)
            fire_idx(g + 2, 1)
            wait_idx(0)
            fire_sca(0)
            return 0
        lax.fori_loop(0, (_GRP - 2) // 2, pair, 0)

        wait_sca(0)
        wait_idx(1)
        fire_sca(1)
        wait_sca(1)

        plsc.subcore_barrier()
        pltpu.sync_copy(acc_sh.at[sl], stage_v)
        pltpu.sync_copy(stage_v, out_hbm.at[pl.ds(c * _NP + s * _RPT, _RPT)])

    return body(er, padch)


def _rsqrt16(d):
    bits = plsc.bitcast(d, jnp.int32)
    y = plsc.bitcast(
        jnp.full((16,), 0x5F3759DF, jnp.int32)
        - lax.shift_right_logical(bits, jnp.full((16,), 1, jnp.int32)),
        _F32)
    hd = 0.5 * d
    for _ in range(3):
        y = y * (1.5 - hd * y * y)
    return y


def _sc_scalar(er, padch, cnt, xflat):
    @functools.partial(
        pl.kernel,
        out_type=[jax.ShapeDtypeStruct((_NC * _NP,), _F32),
                  jax.ShapeDtypeStruct((_NP,), _F32)],
        mesh=_mesh(),
        compiler_params=_sc_params(layout_passes=False),
        scratch_types=[
            pltpu.VMEM((2, _K, _CH), jnp.int32),
            pltpu.VMEM((2, _K, _CH), jnp.int32),
            pltpu.VMEM((2, _K, _CH), _F32),
            pltpu.VMEM((_RPT,), _F32),
            pltpu.VMEM((_RPT,), _F32),
            pltpu.VMEM((_RPT,), _F32),
            pltpu.VMEM_SHARED((_NP,), _F32),
            pltpu.VMEM_SHARED((_NP,), _F32),
            pltpu.SemaphoreType.DMA,
            pltpu.SemaphoreType.DMA,
            pltpu.SemaphoreType.DMA,
            pltpu.SemaphoreType.DMA,
            pltpu.SemaphoreType.DMA,
        ],
    )
    def body(er_hbm, pad_hbm, cnt_hbm, x_hbm, out_hbm, dis_hbm,
             si_v, di_v, vals_v, stage_v, ca_v, cb_v, utab_sh, acc_sh,
             s_i0, s_i1, s_g0, s_g1, s_s):
        c = lax.axis_index("c")
        s = lax.axis_index("s")
        w = c * _NS + s
        base = w * _WCH
        is31 = w == _NW - 1
        sl = pl.ds(s * _RPT, _RPT)

        sem_i = (s_i0, s_i1)
        sem_g = (s_g0, s_g1)
        fire_idx, wait_idx = _edge_loader(
            er_hbm, pad_hbm, 0, base, is31, si_v, di_v, sem_i)

        fire_idx(0, 0)
        fire_idx(1, 1)

        pltpu.sync_copy(cnt_hbm.at[sl], ca_v)
        pltpu.sync_copy(cnt_hbm.at[pl.ds(_NP + s * _RPT, _RPT)], cb_v)
        pltpu.sync_copy(x_hbm.at[sl], stage_v)

        def dcomp(i, _):
            ix = pl.ds(i * 16, 16)
            deg = ca_v[ix] + cb_v[ix] + 1.0
            dis = _rsqrt16(deg)
            ca_v[ix] = dis
            cb_v[ix] = dis * stage_v[ix]
            return 0
        lax.fori_loop(0, _RPT // 16, dcomp, 0)

        pltpu.sync_copy(cb_v, utab_sh.at[sl])

        @pl.when(c == 0)
        def _():
            pltpu.sync_copy(ca_v, dis_hbm.at[sl])

        _zero_flat(stage_v, _RPT)
        pltpu.sync_copy(stage_v, acc_sh.at[sl])
        plsc.subcore_barrier()

        def fire_gat(b):
            for j in range(_K):
                pltpu.async_copy(
                    utab_sh.at[si_v.at[b, j]], vals_v.at[b, j], sem_g[b])

        def wait_gat(b):
            for j in range(_K):
                pltpu.make_async_copy(
                    utab_sh.at[si_v.at[b, j]], vals_v.at[b, j],
                    sem_g[b]).wait()

        def fire_sca(b):
            for j in range(_K):
                pltpu.async_copy(
                    vals_v.at[b, j], acc_sh.at[di_v.at[b, j]], s_s, add=True)

        def wait_sca(b):
            for j in range(_K):
                pltpu.make_async_copy(
                    vals_v.at[b, j], acc_sh.at[di_v.at[b, j]], s_s).wait()

        wait_idx(0)
        fire_gat(0)
        wait_gat(0)
        fire_sca(0)
        wait_idx(1)
        fire_gat(1)

        def steady(g, b):
            nb = 1 - b
            wait_sca(nb)
            fire_idx(g + 1, nb)
            wait_gat(b)
            fire_sca(b)
            wait_idx(nb)
            fire_gat(nb)

        def pair(it, _):
            g = 2 * it + 1
            steady(g, 1)
            steady(g + 1, 0)
            return 0
        lax.fori_loop(0, (_GRP - 2) // 2, pair, 0)

        wait_sca(0)
        wait_gat(1)
        fire_sca(1)
        wait_sca(1)

        plsc.subcore_barrier()
        pltpu.sync_copy(acc_sh.at[sl], stage_v)
        pltpu.sync_copy(stage_v, out_hbm.at[pl.ds(c * _NP + s * _RPT, _RPT)])

    return body(er, padch, cnt, xflat)


def _sc_vec(er, padch, q):
    @functools.partial(
        pl.kernel,
        out_type=jax.ShapeDtypeStruct((_NC * _NP, 16), _F32),
        mesh=_mesh(),
        compiler_params=_sc_params(),
        scratch_types=[
            pltpu.VMEM((2, _K, _CH), jnp.int32),
            pltpu.VMEM((2, _K, _CH), jnp.int32),
            pltpu.VMEM((2, _K * _CH, 16), _F32),
            pltpu.VMEM((_RPT // 4, 16), _F32),
            pltpu.VMEM_SHARED((_NP, 16), _F32),
            pltpu.SemaphoreType.DMA,
            pltpu.SemaphoreType.DMA,
            pltpu.SemaphoreType.DMA,
            pltpu.SemaphoreType.DMA,
            pltpu.SemaphoreType.DMA,
        ],
    )
    def body(er_hbm, pad_hbm, q_hbm, out_hbm,
             si_v, di_v, rows_v, stage_v, acc_sh,
             s_i0, s_i1, s_g0, s_g1, s_s):
        c = lax.axis_index("c")
        s = lax.axis_index("s")
        w = c * _NS + s
        base = w * _WCH
        is31 = w == _NW - 1
        qtr = _RPT // 4
        _zero_rows(stage_v, qtr)
        for r in range(4):
            pltpu.sync_copy(
                stage_v, acc_sh.at[pl.ds(s * _RPT + r * qtr, qtr)])
        plsc.subcore_barrier()

        sem_i = (s_i0, s_i1)
        sem_g = (s_g0, s_g1)
        fire_idx, wait_idx = _edge_loader(
            er_hbm, pad_hbm, 0, base, is31, si_v, di_v, sem_i)

        def fire_gat(b):
            for j in range(_K):
                pltpu.async_copy(
                    q_hbm.at[si_v.at[b, j]],
                    rows_v.at[b, pl.ds(j * _CH, _CH), :], sem_g[b])

        def wait_gat(b):
            for j in range(_K):
                pltpu.make_async_copy(
                    q_hbm.at[si_v.at[b, j]],
                    rows_v.at[b, pl.ds(j * _CH, _CH), :], sem_g[b]).wait()

        def fire_sca(b):
            for j in range(_K):
                pltpu.async_copy(
                    rows_v.at[b, pl.ds(j * _CH, _CH), :],
                    acc_sh.at[di_v.at[b, j]], s_s, add=True)

        def wait_sca(b):
            for j in range(_K):
                pltpu.make_async_copy(
                    rows_v.at[b, pl.ds(j * _CH, _CH), :],
                    acc_sh.at[di_v.at[b, j]], s_s).wait()

        fire_idx(0, 0)
        wait_idx(0)
        fire_gat(0)
        fire_idx(1, 1)
        wait_gat(0)
        fire_sca(0)
        wait_idx(1)
        fire_gat(1)

        def steady(g, b):
            nb = 1 - b
            wait_sca(nb)
            fire_idx(g + 1, nb)
            wait_gat(b)
            fire_sca(b)
            wait_idx(nb)
            fire_gat(nb)

        def pair(it, _):
            g = 2 * it + 1
            steady(g, 1)
            steady(g + 1, 0)
            return 0
        lax.fori_loop(0, (_GRP - 2) // 2, pair, 0)

        wait_sca(0)
        wait_gat(1)
        fire_sca(1)
        wait_sca(1)

        plsc.subcore_barrier()
        for r in range(4):
            pltpu.sync_copy(
                acc_sh.at[pl.ds(s * _RPT + r * qtr, qtr)], stage_v)
            pltpu.sync_copy(
                stage_v,
                out_hbm.at[pl.ds(c * _NP + s * _RPT + r * qtr, qtr), :])

    return body(er, padch, q)


def _tc_prep2(s1p, dis, xp, W1, b1, W2):
    def body(s1a_ref, s1b_ref, dis_ref, x_ref, w1_ref, b1_ref, w2_ref, q_ref):
        dis = dis_ref[...]
        p = dis * (s1a_ref[...] + s1b_ref[...]) + dis * dis * x_ref[...]
        h1 = jnp.maximum(p * w1_ref[...] + b1_ref[...], 0.0)
        t = jnp.dot(h1, w2_ref[...], preferred_element_type=_F32)
        q_ref[...] = dis * t

    grid = (_NB,)
    return pl.pallas_call(
        body,
        grid=grid,
        in_specs=[
            pl.BlockSpec((_BLK, 1), lambda i: (i, 0)),
            pl.BlockSpec((_BLK, 1), lambda i: (_NB + i, 0)),
            pl.BlockSpec((_BLK, 1), lambda i: (i, 0)),
            pl.BlockSpec((_BLK, 1), lambda i: (i, 0)),
            pl.BlockSpec((1, 64), lambda i: (0, 0)),
            pl.BlockSpec((1, 64), lambda i: (0, 0)),
            pl.BlockSpec((64, 16), lambda i: (0, 0)),
        ],
        out_specs=pl.BlockSpec((_BLK, 16), lambda i: (i, 0)),
        out_shape=jax.ShapeDtypeStruct((_NP, 16), _F32),
    )(s1p, s1p, dis, xp, W1, b1, W2)


def _tc_final(s2p, q, dis, batch2d, b2, W3, b3):
    def body(s2a_ref, s2b_ref, q_ref, dis_ref, b_ref, b2_ref, w3_ref, b3_ref,
             out_ref, acc_ref):
        i = pl.program_id(0)

        @pl.when(i == 0)
        def _():
            acc_ref[...] = jnp.zeros((_G, 2), _F32)

        dis = dis_ref[...]
        o2 = dis * (s2a_ref[...] + s2b_ref[...] + q_ref[...]) + b2_ref[...]
        h2 = jnp.maximum(o2, 0.0)
        h3 = jnp.dot(h2, w3_ref[...], preferred_element_type=_F32) \
            + b3_ref[...]
        gids = lax.broadcasted_iota(jnp.int32, (_BLK, _G), 1)
        oh = (b_ref[...] == gids).astype(_F32)
        hcat = jnp.concatenate(
            [h3, jnp.ones((_BLK, 1), _F32)], axis=1)
        acc_ref[...] += lax.dot_general(
            oh, hcat, (((0,), (0,)), ((), ())),
            preferred_element_type=_F32)

        @pl.when(i == _NB - 1)
        def _():
            a = acc_ref[...]
            out_ref[...] = a[:, 0:1] / jnp.maximum(a[:, 1:2], 1.0)

    grid = (_NB,)
    return pl.pallas_call(
        body,
        grid=grid,
        in_specs=[
            pl.BlockSpec((_BLK, 16), lambda i: (i, 0)),
            pl.BlockSpec((_BLK, 16), lambda i: (_NB + i, 0)),
            pl.BlockSpec((_BLK, 16), lambda i: (i, 0)),
            pl.BlockSpec((_BLK, 1), lambda i: (i, 0)),
            pl.BlockSpec((_BLK, 1), lambda i: (i, 0)),
            pl.BlockSpec((1, 16), lambda i: (0, 0)),
            pl.BlockSpec((16, 1), lambda i: (0, 0)),
            pl.BlockSpec((1, 1), lambda i: (0, 0)),
        ],
        out_specs=pl.BlockSpec((_G, 1), lambda i: (0, 0)),
        out_shape=jax.ShapeDtypeStruct((_G, 1), _F32),
        scratch_shapes=[pltpu.VMEM((_G, 2), _F32)],
    )(s2p, s2p, q, dis, batch2d, b2, W3, b3)


def kernel(x, edge_index, batch, W1, b1, W2, b2, W3, b3):
    pad = _NP - _N
    xp = jnp.pad(x[:, 0], (0, pad)).reshape(_NR, 128)
    batch2d = jnp.pad(batch, (0, pad), constant_values=-1)

    er = edge_index.reshape(2, _NCH, _CH)
    padch = (_N + (jnp.arange(_PADCH * _CH, dtype=jnp.int32) % pad)
             ).reshape(_PADCH, _CH)

    cntp = _sc_hist(er, padch)
    x_c = xp.reshape(_NP, 1)

    s1_flat, dis = _sc_scalar(er, padch, cntp, xp.reshape(_NP))
    s1p = s1_flat.reshape(_NC * _NP, 1)
    dis_c = dis.reshape(_NP, 1)
    q = _tc_prep2(s1p, dis_c, x_c, W1, b1.reshape(1, 64), W2)

    s2p = _sc_vec(er, padch, q)
    out = _tc_final(s2p, q, dis_c, batch2d.reshape(_NP, 1),
                    b2.reshape(1, 16), W3, b3.reshape(1, 1))
    return out


# speedup vs baseline: 1.1149x; 1.1143x over previous
import functools

import jax
import jax.numpy as jnp
from jax import lax
from jax.experimental import pallas as pl
from jax.experimental.pallas import tpu as pltpu
from jax.experimental.pallas import tpu_sc as plsc

_N = 50000
_NP = 50176
_E = 3200000
_CH = 128
_NCH = _E // _CH
_K = 8
_G = 128
_NC, _NS = 2, 16
_NW = _NC * _NS
_GRP = 98
_WCH = _GRP * _K
_MAING = (_NCH - 31 * _WCH) // _K
_PADCH = _NW * _WCH - _NCH
_RPT = _NP // _NS
_NB = 49
_BLK = 1024
_NR = _NP // 128

_F32 = jnp.float32


def _mesh():
    return plsc.VectorSubcoreMesh(
        core_axis_name="c", subcore_axis_name="s",
        num_cores=_NC, num_subcores=_NS)


def _sc_params(layout_passes=True):
    return pltpu.CompilerParams(use_tc_tiling_on_sc=False,
                                needs_layout_passes=layout_passes)


def _zero_rows(ref, nrows):
    def body(i, _):
        ref[i] = jnp.zeros((16,), _F32)
        return 0
    lax.fori_loop(0, nrows, body, 0)


def _zero_flat(ref, n):
    def body(i, _):
        ref[pl.ds(i * 16, 16)] = jnp.zeros((16,), _F32)
        return 0
    lax.fori_loop(0, n // 16, body, 0)


def _edge_loader(er_hbm, pad_hbm, row, base, is31, si_v, di_v, sem_i):
    def fire_idx(g, b):
        pad = jnp.logical_and(is31, g >= _MAING)

        @pl.when(jnp.logical_not(pad))
        def _():
            cb = pl.ds(base + g * _K, _K)
            pltpu.async_copy(er_hbm.at[0, cb, :], si_v.at[b], sem_i[b])
            pltpu.async_copy(er_hbm.at[1, cb, :], di_v.at[b], sem_i[b])

        @pl.when(pad)
        def _():
            pb = pl.ds((g - _MAING) * _K, _K)
            pltpu.async_copy(pad_hbm.at[pb, :], si_v.at[b], sem_i[b])
            pltpu.async_copy(pad_hbm.at[pb, :], di_v.at[b], sem_i[b])

    def wait_idx(b):
        cb = pl.ds(0, _K)
        pltpu.make_async_copy(er_hbm.at[0, cb, :], si_v.at[b],
                              sem_i[b]).wait()
        pltpu.make_async_copy(er_hbm.at[1, cb, :], di_v.at[b],
                              sem_i[b]).wait()
    del row
    return fire_idx, wait_idx


def _sc_hist(er, padch):
    @functools.partial(
        pl.kernel,
        out_type=jax.ShapeDtypeStruct((_NC * _NP,), _F32),
        mesh=_mesh(),
        compiler_params=_sc_params(),
        scratch_types=[
            pltpu.VMEM((2, _K, _CH), jnp.int32),
            pltpu.VMEM((_CH,), _F32),
            pltpu.VMEM((_RPT,), _F32),
            pltpu.VMEM_SHARED((_NP,), _F32),
            pltpu.SemaphoreType.DMA,
            pltpu.SemaphoreType.DMA,
            pltpu.SemaphoreType.DMA,
        ],
    )
    def body(er_hbm, pad_hbm, out_hbm, di_v, ones_v, stage_v, acc_sh,
             s_i0, s_i1, s_s):
        c = lax.axis_index("c")
        s = lax.axis_index("s")
        w = c * _NS + s
        base = w * _WCH
        is31 = w == _NW - 1
        sl = pl.ds(s * _RPT, _RPT)
        _zero_flat(stage_v, _RPT)
        for i in range(_CH // 16):
            ones_v[pl.ds(i * 16, 16)] = jnp.ones((16,), _F32)
        pltpu.sync_copy(stage_v, acc_sh.at[sl])
        plsc.subcore_barrier()

        sem_i = (s_i0, s_i1)

        def fire_idx(g, b):
            pad = jnp.logical_and(is31, g >= _MAING)

            @pl.when(jnp.logical_not(pad))
            def _():
                pltpu.async_copy(er_hbm.at[1, pl.ds(base + g * _K, _K), :],
                                 di_v.at[b], sem_i[b])

            @pl.when(pad)
            def _():
                pltpu.async_copy(pad_hbm.at[pl.ds((g - _MAING) * _K, _K), :],
                                 di_v.at[b], sem_i[b])

        def wait_idx(b):
            pltpu.make_async_copy(er_hbm.at[1, pl.ds(0, _K), :], di_v.at[b],
                                  sem_i[b]).wait()

        def fire_sca(b):
            for j in range(_K):
                pltpu.async_copy(ones_v, acc_sh.at[di_v.at[b, j]], s_s,
                                 add=True)

        def wait_sca(b):
            for j in range(_K):
                pltpu.make_async_copy(
                    ones_v, acc_sh.at[di_v.at[b, j]], s_s).wait()

        fire_idx(0, 0)
        fire_idx(1, 1)
        wait_idx(0)
        fire_sca(0)

        def pair(it, _):
            g = 2 * it + 1
            wait_sca(0)
            fire_idx(g + 1, 0)
            wait_idx(1)
            fire_sca(1)
            wait_sca(1)
            fire_idx(g + 2, 1)
            wait_idx(0)
            fire_sca(0)
            return 0
        lax.fori_loop(0, (_GRP - 2) // 2, pair, 0)

        wait_sca(0)
        wait_idx(1)
        fire_sca(1)
        wait_sca(1)

        plsc.subcore_barrier()
        pltpu.sync_copy(acc_sh.at[sl], stage_v)
        pltpu.sync_copy(stage_v, out_hbm.at[pl.ds(c * _NP + s * _RPT, _RPT)])

    return body(er, padch)


def _rsqrt16(d):
    bits = plsc.bitcast(d, jnp.int32)
    y = plsc.bitcast(
        jnp.full((16,), 0x5F3759DF, jnp.int32)
        - lax.shift_right_logical(bits, jnp.full((16,), 1, jnp.int32)),
        _F32)
    hd = 0.5 * d
    for _ in range(3):
        y = y * (1.5 - hd * y * y)
    return y


def _sc_scalar(er, padch, cnt, xflat):
    @functools.partial(
        pl.kernel,
        out_type=[jax.ShapeDtypeStruct((_NC * _NP,), _F32),
                  jax.ShapeDtypeStruct((_NP,), _F32)],
        mesh=_mesh(),
        compiler_params=_sc_params(layout_passes=False),
        scratch_types=[
            pltpu.VMEM((2, _K, _CH), jnp.int32),
            pltpu.VMEM((2, _K, _CH), jnp.int32),
            pltpu.VMEM((2, _K, _CH), _F32),
            pltpu.VMEM((_RPT,), _F32),
            pltpu.VMEM((_RPT,), _F32),
            pltpu.VMEM((_RPT,), _F32),
            pltpu.VMEM_SHARED((_NP,), _F32),
            pltpu.VMEM_SHARED((_NP,), _F32),
            pltpu.SemaphoreType.DMA,
            pltpu.SemaphoreType.DMA,
            pltpu.SemaphoreType.DMA,
            pltpu.SemaphoreType.DMA,
            pltpu.SemaphoreType.DMA,
        ],
    )
    def body(er_hbm, pad_hbm, cnt_hbm, x_hbm, out_hbm, dis_hbm,
             si_v, di_v, vals_v, stage_v, ca_v, cb_v, utab_sh, acc_sh,
             s_i0, s_i1, s_g0, s_g1, s_s):
        c = lax.axis_index("c")
        s = lax.axis_index("s")
        w = c * _NS + s
        base = w * _WCH
        is31 = w == _NW - 1
        sl = pl.ds(s * _RPT, _RPT)

        sem_i = (s_i0, s_i1)
        sem_g = (s_g0, s_g1)
        fire_idx, wait_idx = _edge_loader(
            er_hbm, pad_hbm, 0, base, is31, si_v, di_v, sem_i)

        fire_idx(0, 0)
        fire_idx(1, 1)

        pltpu.sync_copy(cnt_hbm.at[sl], ca_v)
        pltpu.sync_copy(cnt_hbm.at[pl.ds(_NP + s * _RPT, _RPT)], cb_v)
        pltpu.sync_copy(x_hbm.at[sl], stage_v)

        def dcomp(i, _):
            ix = pl.ds(i * 16, 16)
            deg = ca_v[ix] + cb_v[ix] + 1.0
            dis = _rsqrt16(deg)
            ca_v[ix] = dis
            cb_v[ix] = dis * stage_v[ix]
            return 0
        lax.fori_loop(0, _RPT // 16, dcomp, 0)

        pltpu.sync_copy(cb_v, utab_sh.at[sl])

        @pl.when(c == 0)
        def _():
            pltpu.sync_copy(ca_v, dis_hbm.at[sl])

        _zero_flat(stage_v, _RPT)
        pltpu.sync_copy(stage_v, acc_sh.at[sl])
        plsc.subcore_barrier()

        def fire_gat(b):
            for j in range(_K):
                pltpu.async_copy(
                    utab_sh.at[si_v.at[b, j]], vals_v.at[b, j], sem_g[b])

        def wait_gat(b):
            for j in range(_K):
                pltpu.make_async_copy(
                    utab_sh.at[si_v.at[b, j]], vals_v.at[b, j],
                    sem_g[b]).wait()

        def fire_sca(b):
            for j in range(_K):
                pltpu.async_copy(
                    vals_v.at[b, j], acc_sh.at[di_v.at[b, j]], s_s, add=True)

        def wait_sca(b):
            for j in range(_K):
                pltpu.make_async_copy(
                    vals_v.at[b, j], acc_sh.at[di_v.at[b, j]], s_s).wait()

        wait_idx(0)
        fire_gat(0)
        wait_gat(0)
        fire_sca(0)
        wait_idx(1)
        fire_gat(1)

        def steady(g, b):
            nb = 1 - b
            wait_sca(nb)
            fire_idx(g + 1, nb)
            wait_gat(b)
            fire_sca(b)
            wait_idx(nb)
            fire_gat(nb)

        def pair(it, _):
            g = 2 * it + 1
            steady(g, 1)
            steady(g + 1, 0)
            return 0
        lax.fori_loop(0, (_GRP - 2) // 2, pair, 0)

        wait_sca(0)
        wait_gat(1)
        fire_sca(1)
        wait_sca(1)

        plsc.subcore_barrier()
        pltpu.sync_copy(acc_sh.at[sl], stage_v)
        pltpu.sync_copy(stage_v, out_hbm.at[pl.ds(c * _NP + s * _RPT, _RPT)])

    return body(er, padch, cnt, xflat)


def _sc_vec(er, padch, q):
    @functools.partial(
        pl.kernel,
        out_type=jax.ShapeDtypeStruct((_NC * _NP, 16), _F32),
        mesh=_mesh(),
        compiler_params=_sc_params(),
        scratch_types=[
            pltpu.VMEM((2, _K, _CH), jnp.int32),
            pltpu.VMEM((2, _K, _CH), jnp.int32),
            pltpu.VMEM((2, _K * _CH, 16), _F32),
            pltpu.VMEM((_RPT // 4, 16), _F32),
            pltpu.VMEM_SHARED((_NP, 16), _F32),
            pltpu.SemaphoreType.DMA,
            pltpu.SemaphoreType.DMA,
            pltpu.SemaphoreType.DMA,
            pltpu.SemaphoreType.DMA,
            pltpu.SemaphoreType.DMA,
        ],
    )
    def body(er_hbm, pad_hbm, q_hbm, out_hbm,
             si_v, di_v, rows_v, stage_v, acc_sh,
             s_i0, s_i1, s_g0, s_g1, s_s):
        c = lax.axis_index("c")
        s = lax.axis_index("s")
        w = c * _NS + s
        base = w * _WCH
        is31 = w == _NW - 1
        qtr = _RPT // 4
        _zero_rows(stage_v, qtr)
        for r in range(4):
            pltpu.sync_copy(
                stage_v, acc_sh.at[pl.ds(s * _RPT + r * qtr, qtr)])
        plsc.subcore_barrier()

        sem_i = (s_i0, s_i1)
        sem_g = (s_g0, s_g1)
        fire_idx, wait_idx = _edge_loader(
            er_hbm, pad_hbm, 0, base, is31, si_v, di_v, sem_i)

        def fire_gat(b):
            for j in range(_K):
                pltpu.async_copy(
                    q_hbm.at[si_v.at[b, j]],
                    rows_v.at[b, pl.ds(j * _CH, _CH), :], sem_g[b])

        def wait_gat(b):
            for j in range(_K):
                pltpu.make_async_copy(
                    q_hbm.at[si_v.at[b, j]],
                    rows_v.at[b, pl.ds(j * _CH, _CH), :], sem_g[b]).wait()

        def fire_sca(b):
            for j in range(_K):
                pltpu.async_copy(
                    rows_v.at[b, pl.ds(j * _CH, _CH), :],
                    acc_sh.at[di_v.at[b, j]], s_s, add=True)

        def wait_sca(b):
            for j in range(_K):
                pltpu.make_async_copy(
                    rows_v.at[b, pl.ds(j * _CH, _CH), :],
                    acc_sh.at[di_v.at[b, j]], s_s).wait()

        fire_idx(0, 0)
        wait_idx(0)
        fire_gat(0)
        fire_idx(1, 1)
        wait_gat(0)
        fire_sca(0)
        wait_idx(1)
        fire_gat(1)

        def steady(g, b):
            nb = 1 - b
            wait_sca(nb)
            fire_idx(g + 1, nb)
            wait_gat(b)
            fire_sca(b)
            wait_idx(nb)
            fire_gat(nb)

        def pair(it, _):
            g = 2 * it + 1
            steady(g, 1)
            steady(g + 1, 0)
            return 0
        lax.fori_loop(0, (_GRP - 2) // 2, pair, 0)

        wait_sca(0)
        wait_gat(1)
        fire_sca(1)
        wait_sca(1)

        plsc.subcore_barrier()
        for r in range(4):
            pltpu.sync_copy(
                acc_sh.at[pl.ds(s * _RPT + r * qtr, qtr)], stage_v)
            pltpu.sync_copy(
                stage_v,
                out_hbm.at[pl.ds(c * _NP + s * _RPT + r * qtr, qtr), :])

    return body(er, padch, q)


def _tc_prep2(s1p, dis, xp, W1, b1, W2):
    def body(s1a_ref, s1b_ref, dis_ref, x_ref, w1_ref, b1_ref, w2_ref, q_ref):
        dis = dis_ref[...]
        p = dis * (s1a_ref[...] + s1b_ref[...]) + dis * dis * x_ref[...]
        h1 = jnp.maximum(p * w1_ref[...] + b1_ref[...], 0.0)
        t = jnp.dot(h1, w2_ref[...], preferred_element_type=_F32)
        q_ref[...] = dis * t

    grid = (_NB,)
    return pl.pallas_call(
        body,
        grid=grid,
        in_specs=[
            pl.BlockSpec((_BLK, 1), lambda i: (i, 0)),
            pl.BlockSpec((_BLK, 1), lambda i: (_NB + i, 0)),
            pl.BlockSpec((_BLK, 1), lambda i: (i, 0)),
            pl.BlockSpec((_BLK, 1), lambda i: (i, 0)),
            pl.BlockSpec((1, 64), lambda i: (0, 0)),
            pl.BlockSpec((1, 64), lambda i: (0, 0)),
            pl.BlockSpec((64, 16), lambda i: (0, 0)),
        ],
        out_specs=pl.BlockSpec((_BLK, 16), lambda i: (i, 0)),
        out_shape=jax.ShapeDtypeStruct((_NP, 16), _F32),
    )(s1p, s1p, dis, xp, W1, b1, W2)


_NPT = _NP // _NW
_PCH = 112
_NBR = _NP // _PCH
_GP = 144


def _sc_final(s2p, q, dis, batchv, b2, w3):
    @functools.partial(
        pl.kernel,
        out_type=[jax.ShapeDtypeStruct((_NC * _GP, 16), _F32),
                  jax.ShapeDtypeStruct((_NC * _GP,), _F32)],
        mesh=_mesh(),
        compiler_params=_sc_params(layout_passes=False),
        scratch_types=[
            pltpu.VMEM((_NPT, 16), _F32),
            pltpu.VMEM((_NPT, 16), _F32),
            pltpu.VMEM((_NPT, 16), _F32),
            pltpu.VMEM((_NPT,), _F32),
            pltpu.VMEM((_NPT // _PCH, _PCH), jnp.int32),
            pltpu.VMEM((_PCH,), _F32),
            pltpu.VMEM((16,), _F32),
            pltpu.VMEM((16,), _F32),
            pltpu.VMEM((_GP, 16), _F32),
            pltpu.VMEM((_GP,), _F32),
            pltpu.VMEM_SHARED((_GP, 16), _F32),
            pltpu.VMEM_SHARED((_GP,), _F32),
        ],
    )
    def body(s2_hbm, q_hbm, dis_hbm, b_hbm, b2_hbm, w3_hbm,
             pool_hbm, cnt_hbm,
             a_v, b_v, q_v, dis_v, bi_v, ones_v, b2_v, w3_v, stg_v, cstg_v,
             pool_sh, cnt_sh):
        c = lax.axis_index("c")
        s = lax.axis_index("s")
        w = c * _NS + s
        nb = w * _NPT
        sl16 = pl.ds(nb, _NPT)
        pltpu.sync_copy(s2_hbm.at[sl16, :], a_v)
        pltpu.sync_copy(s2_hbm.at[pl.ds(_NP + nb, _NPT), :], b_v)
        pltpu.sync_copy(q_hbm.at[sl16, :], q_v)
        pltpu.sync_copy(dis_hbm.at[sl16], dis_v)
        pltpu.sync_copy(b_hbm.at[pl.ds(w * (_NPT // _PCH), _NPT // _PCH), :],
                        bi_v)
        pltpu.sync_copy(b2_hbm, b2_v)
        pltpu.sync_copy(w3_hbm, w3_v)
        for i in range(_PCH // 16):
            ones_v[pl.ds(i * 16, 16)] = jnp.ones((16,), _F32)

        @pl.when(s == 0)
        def _():
            _zero_rows(stg_v, _GP)
            _zero_flat(cstg_v, _GP)
            pltpu.sync_copy(stg_v, pool_sh)
            pltpu.sync_copy(cstg_v, cnt_sh)

        b2v = b2_v[...]
        w3v = w3_v[...]

        def node(n, _):
            va = a_v[n] + b_v[n] + q_v[n]
            dsp = plsc.load_gather(dis_v, [jnp.full((16,), n, jnp.int32)])
            h2 = jnp.maximum(dsp * va + b2v, 0.0)
            q_v[n] = h2 * w3v
            return 0
        lax.fori_loop(0, _NPT, node, 0)

        plsc.subcore_barrier()
        for j in range(_NPT // _PCH):
            pltpu.sync_copy(q_v.at[pl.ds(j * _PCH, _PCH), :],
                            pool_sh.at[bi_v.at[j]], add=True)
            pltpu.sync_copy(ones_v, cnt_sh.at[bi_v.at[j]], add=True)
        plsc.subcore_barrier()

        @pl.when(s == 0)
        def _():
            pltpu.sync_copy(pool_sh, stg_v)
            pltpu.sync_copy(stg_v, pool_hbm.at[pl.ds(c * _GP, _GP), :])
            pltpu.sync_copy(cnt_sh, cstg_v)
            pltpu.sync_copy(cstg_v, cnt_hbm.at[pl.ds(c * _GP, _GP)])

    return body(s2p, q, dis, batchv, b2, w3)


def kernel(x, edge_index, batch, W1, b1, W2, b2, W3, b3):
    pad = _NP - _N
    xp = jnp.pad(x[:, 0], (0, pad)).reshape(_NR, 128)
    batchv = jnp.pad(batch, (0, pad), constant_values=_G).reshape(_NBR, _PCH)

    er = edge_index.reshape(2, _NCH, _CH)
    padch = (_N + (jnp.arange(_PADCH * _CH, dtype=jnp.int32) % pad)
             ).reshape(_PADCH, _CH)

    cntp = _sc_hist(er, padch)
    x_c = xp.reshape(_NP, 1)

    s1_flat, dis = _sc_scalar(er, padch, cntp, xp.reshape(_NP))
    s1p = s1_flat.reshape(_NC * _NP, 1)
    dis_c = dis.reshape(_NP, 1)
    q = _tc_prep2(s1p, dis_c, x_c, W1, b1.reshape(1, 64), W2)

    s2p = _sc_vec(er, padch, q)
    poolp, cntg = _sc_final(s2p, q, dis, batchv, b2, W3.reshape(16))
    pools = poolp.reshape(_NC, _GP, 16)
    sums = (pools[0, :_G] + pools[1, :_G]).sum(axis=1)
    cnts = cntg.reshape(_NC, _GP)[:, :_G].sum(axis=0)
    out = (sums + cnts * b3[0]) / jnp.maximum(cnts, 1.0)
    return out.reshape(_G, 1)
